# baseline (speedup 1.0000x reference)
import jax
import jax.numpy as jnp
from jax.experimental import pallas as pl

_B, _N, _S = 8, 16384, 5

_K0 = 64467757
_K1 = 2916123636
_K2 = (_K0 ^ _K1 ^ 0x1BD11BDA) & 0xFFFFFFFF

_ROT_A = (13, 15, 26, 6)
_ROT_B = (17, 29, 16, 24)


def _rotl(x, d):
    return (x << jnp.uint32(d)) | (x >> jnp.uint32(32 - d))


def _threefry_kernel(o_ref):
    shape = o_ref.shape
    s = jax.lax.broadcasted_iota(jnp.uint32, shape, 0)
    b = jax.lax.broadcasted_iota(jnp.uint32, shape, 1)
    n = jax.lax.broadcasted_iota(jnp.uint32, shape, 2)
    i = b * jnp.uint32(_N * _S) + n * jnp.uint32(_S) + s

    ks = (jnp.uint32(_K0), jnp.uint32(_K1), jnp.uint32(_K2))
    x0 = jnp.full(shape, _K0, dtype=jnp.uint32)
    x1 = i + jnp.uint32(_K1)

    rots = (_ROT_A, _ROT_B, _ROT_A, _ROT_B, _ROT_A)
    for j in range(5):
        for r in rots[j]:
            x0 = x0 + x1
            x1 = _rotl(x1, r) ^ x0
        x0 = x0 + ks[(j + 1) % 3]
        x1 = x1 + ks[(j + 2) % 3] + jnp.uint32(j + 1)

    o_ref[...] = ((x0 ^ x1) & jnp.uint32(16383)).astype(jnp.int32)


def kernel(x):
    del x
    planes = pl.pallas_call(
        _threefry_kernel,
        out_shape=jax.ShapeDtypeStruct((_S, _B, _N), jnp.int32),
    )()
    return jnp.transpose(planes, (1, 2, 0))


# speedup vs baseline: 1.0934x; 1.0934x over previous
import jax
import jax.numpy as jnp
from jax.experimental import pallas as pl
from jax.experimental.pallas import tpu as pltpu

_B, _N, _S = 8, 16384, 5
_GRID = 8
_BN = _N // _GRID

_K0 = 64467757
_K1 = 2916123636
_K2 = (_K0 ^ _K1 ^ 0x1BD11BDA) & 0xFFFFFFFF

_ROT_A = (13, 15, 26, 6)
_ROT_B = (17, 29, 16, 24)


def _rotl(x, d):
    return (x << jnp.uint32(d)) | (x >> jnp.uint32(32 - d))


def _threefry_kernel(o_ref):
    shape = o_ref.shape
    s = jax.lax.broadcasted_iota(jnp.uint32, shape, 0)
    b = jax.lax.broadcasted_iota(jnp.uint32, shape, 1)
    n = jax.lax.broadcasted_iota(jnp.uint32, shape, 2)
    base = jnp.uint32(pl.program_id(0) * _BN * _S)
    i = base + b * jnp.uint32(_N * _S) + n * jnp.uint32(_S) + s

    ks = (_K0, _K1, _K2)
    x0 = jnp.full(shape, _K0, dtype=jnp.uint32)
    x1 = i + jnp.uint32(_K1)

    rots = (_ROT_A, _ROT_B, _ROT_A, _ROT_B, _ROT_A)
    for j in range(5):
        for r in rots[j]:
            x0 = x0 + x1
            x1 = _rotl(x1, r) ^ x0
        x0 = x0 + jnp.uint32(ks[(j + 1) % 3])
        x1 = x1 + jnp.uint32((ks[(j + 2) % 3] + j + 1) & 0xFFFFFFFF)

    o_ref[...] = ((x0 ^ x1) & jnp.uint32(16383)).astype(jnp.int32)


def kernel(x):
    del x
    planes = pl.pallas_call(
        _threefry_kernel,
        out_shape=jax.ShapeDtypeStruct((_S, _B, _N), jnp.int32),
        grid=(_GRID,),
        out_specs=pl.BlockSpec((_S, _B, _BN), lambda g: (0, 0, g)),
        compiler_params=pltpu.CompilerParams(
            dimension_semantics=("parallel",),
        ),
    )()
    return jnp.transpose(planes, (1, 2, 0))
